# Initial kernel scaffold; baseline (speedup 1.0000x reference)
import jax
import jax.numpy as jnp
from jax.experimental import pallas as pl


def kernel(src_x, dst_x, edge_index, edge_attr, Wsrc, bsrc, Wdst, bdst, We, be, Wattn, battn, Wmsg, bmsg, Wmerge, bmerge, Wout, bout, gamma, beta):
    raise NotImplementedError("write your pallas kernel here")


# speedup vs baseline: 2.6691x; 2.6691x over previous
import functools

import jax
import jax.numpy as jnp
from jax import lax
from jax.experimental import pallas as pl
from jax.experimental.pallas import tpu as pltpu
from jax.experimental.pallas import tpu_sc as plsc

N = 10000
E = 320000
D = 128
H = 4
HD = 32
L16 = 16

NC = 2
NS = 16
C = 40
TPE = E // (NC * NS)
CHUNKS = TPE // C
RPT = 632
N_PAD = NS * RPT
RQ, RTAIL = RPT // C, RPT % C
DG = 1280
DPT = DG // NS
ACC = N_PAD + DG


def _node_tc(src_x, Wsrc, bsrc, WsBD, bsBD, A_s):
    BN = 2000

    def body(sx, ws, bs, wsbd, bsbd, a_s, st_o):
        x = sx[...]
        sh = jnp.dot(x, ws[...], preferred_element_type=jnp.float32) + bs[...]
        sm = jnp.dot(x, wsbd[...], preferred_element_type=jnp.float32) + bsbd[...]
        g = jnp.exp(jnp.dot(jnp.tanh(sh), a_s[...], preferred_element_type=jnp.float32))
        st_o[...] = jnp.concatenate(
            [sm, g, jnp.zeros((BN, D - L16), jnp.float32)], axis=-1)

    row = pl.BlockSpec((BN, D), lambda i: (i, 0))
    wfull = pl.BlockSpec((D, D), lambda i: (0, 0))
    bfull = pl.BlockSpec((1, D), lambda i: (0, 0))
    afull = pl.BlockSpec((D, L16), lambda i: (0, 0))
    return pl.pallas_call(
        body,
        grid=(N // BN,),
        in_specs=[row, wfull, bfull, wfull, bfull, afull],
        out_specs=pl.BlockSpec((BN, 2 * D), lambda i: (i, 0)),
        out_shape=jax.ShapeDtypeStruct((N, 2 * D), jnp.float32),
    )(src_x, Wsrc, bsrc.reshape(1, D), WsBD, bsBD.reshape(1, D), A_s)


def _edge_tc(edge_attr, We, be, WeBD, beBD, A_e, bat):
    BE = 8000
    ED = edge_attr.shape[1]

    def body(ea, we, b_e, webd, bebd, a_e, bt, em_o, ea2_o):
        a = ea[...]
        ef = jnp.dot(a, we[...], preferred_element_type=jnp.float32) + b_e[...]
        em_o[...] = jnp.dot(a, webd[...], preferred_element_type=jnp.float32) + bebd[...]
        ea2_o[...] = jnp.exp(jnp.dot(jnp.tanh(ef), a_e[...], preferred_element_type=jnp.float32) + bt[...])

    row = pl.BlockSpec((BE, ED), lambda i: (i, 0))
    return pl.pallas_call(
        body,
        grid=(E // BE,),
        in_specs=[row,
                  pl.BlockSpec((ED, D), lambda i: (0, 0)),
                  pl.BlockSpec((1, D), lambda i: (0, 0)),
                  pl.BlockSpec((ED, D), lambda i: (0, 0)),
                  pl.BlockSpec((1, D), lambda i: (0, 0)),
                  pl.BlockSpec((D, L16), lambda i: (0, 0)),
                  pl.BlockSpec((1, L16), lambda i: (0, 0))],
        out_specs=[pl.BlockSpec((BE, D), lambda i: (i, 0)),
                   pl.BlockSpec((BE, L16), lambda i: (i, 0))],
        out_shape=[
            jax.ShapeDtypeStruct((E, D), jnp.float32),
            jax.ShapeDtypeStruct((E, L16), jnp.float32),
        ],
    )(edge_attr, We, be.reshape(1, D), WeBD, beBD.reshape(1, D), A_e, bat)


def _sc_edge(src_idx, dst_idx, didx8n, dmod16, eea, st, e_m):
    mesh = plsc.VectorSubcoreMesh(core_axis_name="c", subcore_axis_name="s")

    @functools.partial(
        pl.kernel,
        out_type=(jax.ShapeDtypeStruct((NC, N_PAD, D), jnp.float32),
                  jax.ShapeDtypeStruct((NC, DG, D), jnp.float32)),
        mesh=mesh,
        scratch_types=[
            pltpu.VMEM((C,), jnp.int32),
            pltpu.VMEM((C,), jnp.int32),
            pltpu.VMEM((C,), jnp.int32),
            pltpu.VMEM((C, L16), jnp.float32),
            pltpu.VMEM((C, L16), jnp.float32),
            pltpu.VMEM((C, 2 * D), jnp.float32),
            pltpu.VMEM((C, D), jnp.float32),
            pltpu.VMEM((C, D), jnp.float32),
            pltpu.VMEM_SHARED((ACC, D), jnp.float32),
            pltpu.SemaphoreType.DMA,
        ],
    )
    def k(sidx_hbm, didx_hbm, didx8_hbm, dmod_hbm, eea_hbm, st_hbm, em_hbm,
          num_out, den_out,
          sidx_v, didx_v, didx8_v, eea_v, dmod_v, st_v, emb_v, dbuf_v,
          acc_sh, sem):
        c = lax.axis_index("c")
        s = lax.axis_index("s")
        zero16 = jnp.zeros((L16,), jnp.float32)

        def zrow(i, cc):
            for l in range(D // L16):
                emb_v[i, pl.ds(l * L16, L16)] = zero16
            return cc
        lax.fori_loop(0, C, zrow, 0)
        zbase = s * (ACC // NS)
        for q in range(ACC // NS // C):
            pltpu.sync_copy(emb_v, acc_sh.at[pl.ds(zbase + q * C, C)])
        zt = ACC // NS - (ACC // NS // C) * C
        if zt:
            pltpu.sync_copy(emb_v.at[pl.ds(0, zt)],
                            acc_sh.at[pl.ds(zbase + (ACC // NS // C) * C, zt)])
        plsc.subcore_barrier()

        ebase = (c * NS + s) * TPE

        def chunk(ci, cc):
            off = ebase + ci * C
            pltpu.sync_copy(sidx_hbm.at[pl.ds(off, C)], sidx_v)
            pltpu.sync_copy(didx_hbm.at[pl.ds(off, C)], didx_v)
            pltpu.sync_copy(didx8_hbm.at[pl.ds(off, C)], didx8_v)
            pltpu.sync_copy(eea_hbm.at[pl.ds(off, C)], eea_v)
            pltpu.sync_copy(dmod_hbm.at[pl.ds(off, C)], dmod_v)
            pltpu.sync_copy(em_hbm.at[pl.ds(off, C)], emb_v)
            pltpu.async_copy(st_hbm.at[sidx_v], st_v, sem).wait()

            def row(i, rc):
                ex16 = st_v[i, pl.ds(D, L16)] * eea_v[i, :]
                dm = dmod_v[i, :]
                for h in range(H):
                    spl = jnp.broadcast_to(ex16[4 * h], (L16,))
                    for l in (2 * h, 2 * h + 1):
                        sl = pl.ds(l * L16, L16)
                        emb_v[i, sl] = (st_v[i, sl] + emb_v[i, sl]) * spl
                for l in range(D // L16):
                    dbuf_v[i, pl.ds(l * L16, L16)] = jnp.where(
                        dm == float(l), ex16, 0.0)
                return rc
            lax.fori_loop(0, C, row, 0)
            pltpu.sync_copy(emb_v, acc_sh.at[didx_v], add=True)
            pltpu.sync_copy(dbuf_v, acc_sh.at[didx8_v], add=True)
            return cc
        lax.fori_loop(0, CHUNKS, chunk, 0)
        plsc.subcore_barrier()

        nbase = s * RPT
        for q in range(RQ):
            pltpu.sync_copy(acc_sh.at[pl.ds(nbase + q * C, C)], emb_v)
            pltpu.sync_copy(emb_v, num_out.at[c, pl.ds(nbase + q * C, C)])
        pltpu.sync_copy(acc_sh.at[pl.ds(nbase + RQ * C, RTAIL)],
                        emb_v.at[pl.ds(0, RTAIL)])
        pltpu.sync_copy(emb_v.at[pl.ds(0, RTAIL)],
                        num_out.at[c, pl.ds(nbase + RQ * C, RTAIL)])
        dbase = s * DPT
        for q in range(DPT // C):
            pltpu.sync_copy(acc_sh.at[pl.ds(N_PAD + dbase + q * C, C)], dbuf_v)
            pltpu.sync_copy(dbuf_v, den_out.at[c, pl.ds(dbase + q * C, C)])

    return k(src_idx, dst_idx, didx8n, dmod16, eea, st, e_m)


def _tail_tc(dst_x, num0, num1, den0, den1, Wo1, B2, c2, R, gamma, beta):
    BN = 2000

    def body(dx, n0, n1, d0, d1, wo1, b2, c2r, r, g, b, out):
        x = dx[...]
        num = n0[...] + n1[...]
        den = d0[...] + d1[...]
        den_bc = jnp.dot(den, r[...], preferred_element_type=jnp.float32)
        agg = num / jnp.where(den_bc > 0, den_bc, 1.0)
        res = (x + jnp.dot(x, wo1[...], preferred_element_type=jnp.float32)
               + jnp.dot(agg, b2[...], preferred_element_type=jnp.float32) + c2r[...])
        mu = jnp.mean(res, axis=-1, keepdims=True)
        cen = res - mu
        var = jnp.mean(cen * cen, axis=-1, keepdims=True)
        out[...] = cen * jax.lax.rsqrt(var + 1e-5) * g[...] + b[...]

    row = pl.BlockSpec((BN, D), lambda i: (i, 0))
    att = pl.BlockSpec((BN, L16), lambda i: (i, 0))
    wfull = pl.BlockSpec((D, D), lambda i: (0, 0))
    bfull = pl.BlockSpec((1, D), lambda i: (0, 0))
    return pl.pallas_call(
        body,
        grid=(N // BN,),
        in_specs=[row, row, row, att, att, wfull, wfull, bfull,
                  pl.BlockSpec((L16, D), lambda i: (0, 0)), bfull, bfull],
        out_specs=row,
        out_shape=jax.ShapeDtypeStruct((N, D), jnp.float32),
    )(dst_x, num0, num1, den0, den1, Wo1, B2, c2.reshape(1, D), R,
      gamma.reshape(1, D), beta.reshape(1, D))


def kernel(src_x, dst_x, edge_index, edge_attr, Wsrc, bsrc, Wdst, bdst, We, be,
           Wattn, battn, Wmsg, bmsg, Wmerge, bmerge, Wout, bout, gamma, beta):
    BD = jax.scipy.linalg.block_diag(*([Wmsg] * H))
    d_ar = jnp.arange(D)
    j_ar = jnp.arange(L16)
    headmask = (d_ar[:, None] // HD) == (j_ar[None, :] // 4)

    def attn_fold(off):
        return jnp.where(headmask, Wattn[off + (d_ar % HD), 0][:, None], 0.0)
    A_s, A_d, A_e = attn_fold(0), attn_fold(HD), attn_fold(2 * HD)
    WsBD = Wsrc @ BD
    bsBD = bsrc @ BD
    WeBD = We @ BD
    beBD = be @ BD + jnp.tile(bmsg, H)
    bat = jnp.broadcast_to(battn[0], (1, L16)).astype(jnp.float32)
    R = jnp.where(j_ar[:, None] == 4 * (d_ar[None, :] // HD), 1.0, 0.0)
    Wm_t = jnp.tile(Wmerge, (H, 1)) / H
    B2 = Wm_t @ Wout[D:]
    c2 = bmerge @ Wout[D:] + bout

    st = _node_tc(src_x, Wsrc, bsrc, WsBD, bsBD, A_s)
    e_m, eea = _edge_tc(edge_attr, We, be, WeBD, beBD, A_e, bat)

    src_idx, dst_idx = edge_index[0], edge_index[1]
    didx8n = N_PAD + dst_idx // 8
    dmod16 = jnp.broadcast_to(
        (dst_idx % 8).astype(jnp.float32)[:, None], (E, L16))
    num_p, den_p = _sc_edge(src_idx, dst_idx, didx8n, dmod16, eea, st, e_m)
    den_r = den_p.reshape(NC, DG * 8, L16)

    return _tail_tc(dst_x, num_p[0, :N], num_p[1, :N],
                    den_r[0, :N], den_r[1, :N],
                    Wout[:D], B2, c2, R, gamma, beta)


# speedup vs baseline: 3.5429x; 1.3274x over previous
import functools

import jax
import jax.numpy as jnp
from jax import lax
from jax.experimental import pallas as pl
from jax.experimental.pallas import tpu as pltpu
from jax.experimental.pallas import tpu_sc as plsc

N = 10000
E = 320000
D = 128
H = 4
HD = 32
L16 = 16

NC = 2
NS = 16
C = 40
TPE = E // (NC * NS)
CHUNKS = TPE // C
RPT = 632
N_PAD = NS * RPT
RQ, RTAIL = RPT // C, RPT % C
DG = 1280
DPT = DG // NS
ACC = N_PAD + DG


def _node_tc(src_x, Wsrc, bsrc, WsBD, bsBD, A_s):
    BN = 2000

    def body(sx, ws, bs, wsbd, bsbd, a_s, st_o):
        x = sx[...]
        sh = jnp.dot(x, ws[...], preferred_element_type=jnp.float32) + bs[...]
        sm = jnp.dot(x, wsbd[...], preferred_element_type=jnp.float32) + bsbd[...]
        g = jnp.exp(jnp.dot(jnp.tanh(sh), a_s[...], preferred_element_type=jnp.float32))
        st_o[...] = jnp.concatenate(
            [sm, g, jnp.zeros((BN, D - L16), jnp.float32)], axis=-1)

    row = pl.BlockSpec((BN, D), lambda i: (i, 0))
    wfull = pl.BlockSpec((D, D), lambda i: (0, 0))
    bfull = pl.BlockSpec((1, D), lambda i: (0, 0))
    afull = pl.BlockSpec((D, L16), lambda i: (0, 0))
    return pl.pallas_call(
        body,
        grid=(N // BN,),
        in_specs=[row, wfull, bfull, wfull, bfull, afull],
        out_specs=pl.BlockSpec((BN, 2 * D), lambda i: (i, 0)),
        out_shape=jax.ShapeDtypeStruct((N, 2 * D), jnp.float32),
    )(src_x, Wsrc, bsrc.reshape(1, D), WsBD, bsBD.reshape(1, D), A_s)


def _edge_tc(edge_attr, dmod16, We, be, WeBD, beBD, A_e, bat):
    BE = 8000
    ED = edge_attr.shape[1]

    def body(ea, dm, we, b_e, webd, bebd, a_e, bt, ep_o):
        a = ea[...]
        ef = jnp.dot(a, we[...], preferred_element_type=jnp.float32) + b_e[...]
        em = jnp.dot(a, webd[...], preferred_element_type=jnp.float32) + bebd[...]
        eea = jnp.exp(jnp.dot(jnp.tanh(ef), a_e[...],
                              preferred_element_type=jnp.float32) + bt[...])
        ep_o[...] = jnp.concatenate(
            [em, eea, dm[...], jnp.zeros((BE, D - 2 * L16), jnp.float32)], axis=-1)

    row = pl.BlockSpec((BE, ED), lambda i: (i, 0))
    return pl.pallas_call(
        body,
        grid=(E // BE,),
        in_specs=[row,
                  pl.BlockSpec((BE, L16), lambda i: (i, 0)),
                  pl.BlockSpec((ED, D), lambda i: (0, 0)),
                  pl.BlockSpec((1, D), lambda i: (0, 0)),
                  pl.BlockSpec((ED, D), lambda i: (0, 0)),
                  pl.BlockSpec((1, D), lambda i: (0, 0)),
                  pl.BlockSpec((D, L16), lambda i: (0, 0)),
                  pl.BlockSpec((1, L16), lambda i: (0, 0))],
        out_specs=pl.BlockSpec((BE, 2 * D), lambda i: (i, 0)),
        out_shape=jax.ShapeDtypeStruct((E, 2 * D), jnp.float32),
    )(edge_attr, dmod16, We, be.reshape(1, D), WeBD, beBD.reshape(1, D), A_e, bat)


def _sc_edge(src_idx, idxcat, epack, st):
    mesh = plsc.VectorSubcoreMesh(core_axis_name="c", subcore_axis_name="s")
    EPW = 2 * D

    @functools.partial(
        pl.kernel,
        out_type=(jax.ShapeDtypeStruct((NC, N_PAD, D), jnp.float32),
                  jax.ShapeDtypeStruct((NC, DG, D), jnp.float32)),
        mesh=mesh,
        scratch_types=[
            pltpu.VMEM((C,), jnp.int32),
            pltpu.VMEM((2 * C,), jnp.int32),
            pltpu.VMEM((C, EPW), jnp.float32),
            pltpu.VMEM((C, 2 * D), jnp.float32),
            pltpu.VMEM((2 * C, D), jnp.float32),
            pltpu.VMEM_SHARED((ACC, D), jnp.float32),
            pltpu.SemaphoreType.DMA,
            pltpu.SemaphoreType.DMA,
        ],
    )
    def k(sidx_hbm, idxcat_hbm, ep_hbm, st_hbm,
          num_out, den_out,
          sidx_v, idxcat_v, ep_v, st_v, wd_v,
          acc_sh, sem_in, sem_g):
        c = lax.axis_index("c")
        s = lax.axis_index("s")
        zero16 = jnp.zeros((L16,), jnp.float32)

        def zrow(i, cc):
            for l in range(D // L16):
                wd_v[i, pl.ds(l * L16, L16)] = zero16
            return cc
        lax.fori_loop(0, 2 * C, zrow, 0)
        acc_pt = ACC // NS
        zbase = s * acc_pt
        for q in range(acc_pt // (2 * C)):
            pltpu.sync_copy(wd_v, acc_sh.at[pl.ds(zbase + q * 2 * C, 2 * C)])
        zt = acc_pt - (acc_pt // (2 * C)) * (2 * C)
        if zt:
            pltpu.sync_copy(wd_v.at[pl.ds(0, zt)],
                            acc_sh.at[pl.ds(zbase + acc_pt - zt, zt)])
        plsc.subcore_barrier()

        tbase = (c * NS + s) * CHUNKS

        def chunk(ci, cc):
            g = tbase + ci
            off = g * C
            d1 = pltpu.async_copy(sidx_hbm.at[pl.ds(off, C)], sidx_v, sem_in)
            d2 = pltpu.async_copy(idxcat_hbm.at[pl.ds(g * 2 * C, 2 * C)], idxcat_v, sem_in)
            d3 = pltpu.async_copy(ep_hbm.at[pl.ds(off, C)], ep_v, sem_in)
            d1.wait()
            d2.wait()
            d3.wait()
            pltpu.async_copy(st_hbm.at[sidx_v], st_v, sem_g).wait()

            def row(i, rc):
                ex16 = st_v[i, pl.ds(D, L16)] * ep_v[i, pl.ds(D, L16)]
                dm = ep_v[i, pl.ds(D + L16, L16)]
                for h in range(H):
                    spl = jnp.broadcast_to(ex16[4 * h], (L16,))
                    for l in (2 * h, 2 * h + 1):
                        sl = pl.ds(l * L16, L16)
                        wd_v[i, sl] = (st_v[i, sl] + ep_v[i, sl]) * spl
                for l in range(D // L16):
                    wd_v[C + i, pl.ds(l * L16, L16)] = jnp.where(
                        dm == float(l), ex16, 0.0)
                return rc
            lax.fori_loop(0, C, row, 0, unroll=4)
            pltpu.sync_copy(wd_v, acc_sh.at[idxcat_v], add=True)
            return cc
        lax.fori_loop(0, CHUNKS, chunk, 0)
        plsc.subcore_barrier()

        nbase = s * RPT
        for q in range(RPT // (2 * C)):
            pltpu.sync_copy(acc_sh.at[pl.ds(nbase + q * 2 * C, 2 * C)], wd_v)
            pltpu.sync_copy(wd_v, num_out.at[c, pl.ds(nbase + q * 2 * C, 2 * C)])
        rt = RPT - (RPT // (2 * C)) * (2 * C)
        if rt:
            pltpu.sync_copy(acc_sh.at[pl.ds(nbase + RPT - rt, rt)],
                            wd_v.at[pl.ds(0, rt)])
            pltpu.sync_copy(wd_v.at[pl.ds(0, rt)],
                            num_out.at[c, pl.ds(nbase + RPT - rt, rt)])
        dbase = s * DPT
        pltpu.sync_copy(acc_sh.at[pl.ds(N_PAD + dbase, DPT)],
                        wd_v.at[pl.ds(0, DPT)])
        pltpu.sync_copy(wd_v.at[pl.ds(0, DPT)],
                        den_out.at[c, pl.ds(dbase, DPT)])

    return k(src_idx, idxcat, epack, st)


def _tail_tc(dst_x, num0, num1, den0, den1, Wo1, B2, c2, R, gamma, beta):
    BN = 2000

    def body(dx, n0, n1, d0, d1, wo1, b2, c2r, r, g, b, out):
        x = dx[...]
        num = n0[...] + n1[...]
        den = d0[...] + d1[...]
        den_bc = jnp.dot(den, r[...], preferred_element_type=jnp.float32)
        agg = num / jnp.where(den_bc > 0, den_bc, 1.0)
        res = (x + jnp.dot(x, wo1[...], preferred_element_type=jnp.float32)
               + jnp.dot(agg, b2[...], preferred_element_type=jnp.float32) + c2r[...])
        mu = jnp.mean(res, axis=-1, keepdims=True)
        cen = res - mu
        var = jnp.mean(cen * cen, axis=-1, keepdims=True)
        out[...] = cen * jax.lax.rsqrt(var + 1e-5) * g[...] + b[...]

    row = pl.BlockSpec((BN, D), lambda i: (i, 0))
    att = pl.BlockSpec((BN, L16), lambda i: (i, 0))
    wfull = pl.BlockSpec((D, D), lambda i: (0, 0))
    bfull = pl.BlockSpec((1, D), lambda i: (0, 0))
    return pl.pallas_call(
        body,
        grid=(N // BN,),
        in_specs=[row, row, row, att, att, wfull, wfull, bfull,
                  pl.BlockSpec((L16, D), lambda i: (0, 0)), bfull, bfull],
        out_specs=row,
        out_shape=jax.ShapeDtypeStruct((N, D), jnp.float32),
    )(dst_x, num0, num1, den0, den1, Wo1, B2, c2.reshape(1, D), R,
      gamma.reshape(1, D), beta.reshape(1, D))


def kernel(src_x, dst_x, edge_index, edge_attr, Wsrc, bsrc, Wdst, bdst, We, be,
           Wattn, battn, Wmsg, bmsg, Wmerge, bmerge, Wout, bout, gamma, beta):
    BD = jax.scipy.linalg.block_diag(*([Wmsg] * H))
    d_ar = jnp.arange(D)
    j_ar = jnp.arange(L16)
    headmask = (d_ar[:, None] // HD) == (j_ar[None, :] // 4)

    def attn_fold(off):
        return jnp.where(headmask, Wattn[off + (d_ar % HD), 0][:, None], 0.0)
    A_s, A_d, A_e = attn_fold(0), attn_fold(HD), attn_fold(2 * HD)
    WsBD = Wsrc @ BD
    bsBD = bsrc @ BD
    WeBD = We @ BD
    beBD = be @ BD + jnp.tile(bmsg, H)
    bat = jnp.broadcast_to(battn[0], (1, L16)).astype(jnp.float32)
    R = jnp.where(j_ar[:, None] == 4 * (d_ar[None, :] // HD), 1.0, 0.0)
    Wm_t = jnp.tile(Wmerge, (H, 1)) / H
    B2 = Wm_t @ Wout[D:]
    c2 = bmerge @ Wout[D:] + bout

    src_idx, dst_idx = edge_index[0], edge_index[1]
    didx8n = N_PAD + dst_idx // 8
    dmod16 = jnp.broadcast_to(
        (dst_idx % 8).astype(jnp.float32)[:, None], (E, L16))
    st = _node_tc(src_x, Wsrc, bsrc, WsBD, bsBD, A_s)
    epack = _edge_tc(edge_attr, dmod16, We, be, WeBD, beBD, A_e, bat)
    G = E // C
    idxcat = jnp.concatenate(
        [dst_idx.reshape(G, C), didx8n.reshape(G, C)], axis=1).reshape(-1)

    num_p, den_p = _sc_edge(src_idx, idxcat, epack, st)
    den_r = den_p.reshape(NC, DG * 8, L16)

    return _tail_tc(dst_x, num_p[0, :N], num_p[1, :N],
                    den_r[0, :N], den_r[1, :N],
                    Wout[:D], B2, c2, R, gamma, beta)


# speedup vs baseline: 3.9614x; 1.1181x over previous
import functools

import jax
import jax.numpy as jnp
from jax import lax
from jax.experimental import pallas as pl
from jax.experimental.pallas import tpu as pltpu
from jax.experimental.pallas import tpu_sc as plsc

N = 10000
E = 320000
D = 128
H = 4
HD = 32
L16 = 16

NC = 2
NS = 16
C = 40
TPE = E // (NC * NS)
CHUNKS = TPE // C
RPT = 632
N_PAD = NS * RPT
RQ, RTAIL = RPT // C, RPT % C
DG = 320
ACC = 10496


def _node_tc(src_x, Wsrc, bsrc, WsBD, bsBD, A_s):
    BN = 2000

    def body(sx, ws, bs, wsbd, bsbd, a_s, st_o):
        x = sx[...]
        sh = jnp.dot(x, ws[...], preferred_element_type=jnp.float32) + bs[...]
        sm = jnp.dot(x, wsbd[...], preferred_element_type=jnp.float32) + bsbd[...]
        g = jnp.exp(jnp.dot(jnp.tanh(sh), a_s[...], preferred_element_type=jnp.float32))
        st_o[...] = jnp.concatenate(
            [sm, g, jnp.zeros((BN, D - L16), jnp.float32)], axis=-1)

    row = pl.BlockSpec((BN, D), lambda i: (i, 0))
    wfull = pl.BlockSpec((D, D), lambda i: (0, 0))
    bfull = pl.BlockSpec((1, D), lambda i: (0, 0))
    afull = pl.BlockSpec((D, L16), lambda i: (0, 0))
    return pl.pallas_call(
        body,
        grid=(N // BN,),
        in_specs=[row, wfull, bfull, wfull, bfull, afull],
        out_specs=pl.BlockSpec((BN, 2 * D), lambda i: (i, 0)),
        out_shape=jax.ShapeDtypeStruct((N, 2 * D), jnp.float32),
    )(src_x, Wsrc, bsrc.reshape(1, D), WsBD, bsBD.reshape(1, D), A_s)


def _edge_tc(edge_attr, dmod16, We, be, WeBD, beBD, A_e, bat):
    BE = 8000
    ED = edge_attr.shape[1]

    def body(ea, dm, we, b_e, webd, bebd, a_e, bt, ep_o):
        a = ea[...]
        ef = jnp.dot(a, we[...], preferred_element_type=jnp.float32) + b_e[...]
        em = jnp.dot(a, webd[...], preferred_element_type=jnp.float32) + bebd[...]
        eea = jnp.exp(jnp.dot(jnp.tanh(ef), a_e[...],
                              preferred_element_type=jnp.float32) + bt[...])
        ep_o[...] = jnp.concatenate([em, eea, dm[...]], axis=-1)

    row = pl.BlockSpec((BE, ED), lambda i: (i, 0))
    return pl.pallas_call(
        body,
        grid=(E // BE,),
        in_specs=[row,
                  pl.BlockSpec((BE, L16), lambda i: (i, 0)),
                  pl.BlockSpec((ED, D), lambda i: (0, 0)),
                  pl.BlockSpec((1, D), lambda i: (0, 0)),
                  pl.BlockSpec((ED, D), lambda i: (0, 0)),
                  pl.BlockSpec((1, D), lambda i: (0, 0)),
                  pl.BlockSpec((D, L16), lambda i: (0, 0)),
                  pl.BlockSpec((1, L16), lambda i: (0, 0))],
        out_specs=pl.BlockSpec((BE, D + 2 * L16), lambda i: (i, 0)),
        out_shape=jax.ShapeDtypeStruct((E, D + 2 * L16), jnp.float32),
    )(edge_attr, dmod16, We, be.reshape(1, D), WeBD, beBD.reshape(1, D), A_e, bat)


def _sc_edge(src_idx, idxcat, epack, st):
    mesh = plsc.VectorSubcoreMesh(core_axis_name="c", subcore_axis_name="s")
    EPW = D + 2 * L16

    @functools.partial(
        pl.kernel,
        out_type=(jax.ShapeDtypeStruct((NC, N_PAD, D), jnp.float32),
                  jax.ShapeDtypeStruct((NC, DG, D), jnp.float32)),
        mesh=mesh,
        scratch_types=[
            pltpu.VMEM((C,), jnp.int32),
            pltpu.VMEM((C,), jnp.int32),
            pltpu.VMEM((2 * C,), jnp.int32),
            pltpu.VMEM((2 * C,), jnp.int32),
            pltpu.VMEM((C, EPW), jnp.float32),
            pltpu.VMEM((C, EPW), jnp.float32),
            pltpu.VMEM((C, 2 * D), jnp.float32),
            pltpu.VMEM((2 * C, D), jnp.float32),
            pltpu.VMEM_SHARED((ACC, D), jnp.float32),
            pltpu.SemaphoreType.DMA,
            pltpu.SemaphoreType.DMA,
            pltpu.SemaphoreType.DMA,
        ],
    )
    def k(sidx_hbm, idxcat_hbm, ep_hbm, st_hbm,
          num_out, den_out,
          sidx_v0, sidx_v1, idxcat_v0, idxcat_v1, ep_v0, ep_v1, st_v, wd_v,
          acc_sh, sem_a, sem_b, sem_g):
        c = lax.axis_index("c")
        s = lax.axis_index("s")
        zero16 = jnp.zeros((L16,), jnp.float32)

        def zrow(i, cc):
            for l in range(D // L16):
                wd_v[i, pl.ds(l * L16, L16)] = zero16
            return cc
        lax.fori_loop(0, 2 * C, zrow, 0)
        acc_pt = ACC // NS
        zbase = s * acc_pt
        for q in range(acc_pt // (2 * C)):
            pltpu.sync_copy(wd_v, acc_sh.at[pl.ds(zbase + q * 2 * C, 2 * C)])
        zt = acc_pt - (acc_pt // (2 * C)) * (2 * C)
        if zt:
            pltpu.sync_copy(wd_v.at[pl.ds(0, zt)],
                            acc_sh.at[pl.ds(zbase + acc_pt - zt, zt)])
        plsc.subcore_barrier()

        tbase = (c * NS + s) * CHUNKS
        NPAIR = CHUNKS // 2

        def issue(off, sv, iv, ev, sem):
            pltpu.async_copy(sidx_hbm.at[pl.ds(off, C)], sv, sem)
            pltpu.async_copy(idxcat_hbm.at[pl.ds(2 * off, 2 * C)], iv, sem)
            pltpu.async_copy(ep_hbm.at[pl.ds(off, C)], ev, sem)

        def drain(sv, iv, ev, sem):
            pltpu.make_async_copy(sidx_hbm.at[pl.ds(0, C)], sv, sem).wait()
            pltpu.make_async_copy(idxcat_hbm.at[pl.ds(0, 2 * C)], iv, sem).wait()
            pltpu.make_async_copy(ep_hbm.at[pl.ds(0, C)], ev, sem).wait()

        def process(sv, iv, ev):
            pltpu.async_copy(st_hbm.at[sv], st_v, sem_g).wait()

            def row(i, rc):
                ex16 = st_v[i, pl.ds(D, L16)] * ev[i, pl.ds(D, L16)]
                dm = ev[i, pl.ds(D + L16, L16)]
                for h in range(H):
                    spl = jnp.broadcast_to(ex16[h], (L16,))
                    for l in (2 * h, 2 * h + 1):
                        sl = pl.ds(l * L16, L16)
                        wd_v[i, sl] = (st_v[i, sl] + ev[i, sl]) * spl
                for l in range(D // L16):
                    wd_v[C + i, pl.ds(l * L16, L16)] = jnp.where(
                        dm == float(4 * l), ex16, 0.0)
                return rc
            lax.fori_loop(0, C, row, 0, unroll=4)
            pltpu.sync_copy(wd_v, acc_sh.at[iv], add=True)

        issue(tbase * C, sidx_v0, idxcat_v0, ep_v0, sem_a)

        def pair(p, cc):
            offA = (tbase + 2 * p) * C
            drain(sidx_v0, idxcat_v0, ep_v0, sem_a)
            issue(offA + C, sidx_v1, idxcat_v1, ep_v1, sem_b)
            process(sidx_v0, idxcat_v0, ep_v0)
            drain(sidx_v1, idxcat_v1, ep_v1, sem_b)

            @pl.when(p < NPAIR - 1)
            def _():
                issue(offA + 2 * C, sidx_v0, idxcat_v0, ep_v0, sem_a)
            process(sidx_v1, idxcat_v1, ep_v1)
            return cc
        lax.fori_loop(0, NPAIR, pair, 0)
        plsc.subcore_barrier()

        nbase = s * RPT
        for q in range(RPT // (2 * C)):
            pltpu.sync_copy(acc_sh.at[pl.ds(nbase + q * 2 * C, 2 * C)], wd_v)
            pltpu.sync_copy(wd_v, num_out.at[c, pl.ds(nbase + q * 2 * C, 2 * C)])
        rt = RPT - (RPT // (2 * C)) * (2 * C)
        if rt:
            pltpu.sync_copy(acc_sh.at[pl.ds(nbase + RPT - rt, rt)],
                            wd_v.at[pl.ds(0, rt)])
            pltpu.sync_copy(wd_v.at[pl.ds(0, rt)],
                            num_out.at[c, pl.ds(nbase + RPT - rt, rt)])
        @pl.when(s < DG // (2 * C))
        def _():
            dbase = s * 2 * C
            pltpu.sync_copy(acc_sh.at[pl.ds(N_PAD + dbase, 2 * C)], wd_v)
            pltpu.sync_copy(wd_v, den_out.at[c, pl.ds(dbase, 2 * C)])

    return k(src_idx, idxcat, epack, st)


def _tail_tc(dst_x, num0, num1, den0, den1, Wo1, B2, c2, R, gamma, beta):
    BN = 2000

    def body(dx, n0, n1, d0, d1, wo1, b2, c2r, r, g, b, out):
        x = dx[...]
        num = n0[...] + n1[...]
        den = d0[...] + d1[...]
        den_bc = jnp.dot(den, r[...], preferred_element_type=jnp.float32)
        agg = num / jnp.where(den_bc > 0, den_bc, 1.0)
        res = (x + jnp.dot(x, wo1[...], preferred_element_type=jnp.float32)
               + jnp.dot(agg, b2[...], preferred_element_type=jnp.float32) + c2r[...])
        mu = jnp.mean(res, axis=-1, keepdims=True)
        cen = res - mu
        var = jnp.mean(cen * cen, axis=-1, keepdims=True)
        out[...] = cen * jax.lax.rsqrt(var + 1e-5) * g[...] + b[...]

    row = pl.BlockSpec((BN, D), lambda i: (i, 0))
    att = pl.BlockSpec((BN, H), lambda i: (i, 0))
    wfull = pl.BlockSpec((D, D), lambda i: (0, 0))
    bfull = pl.BlockSpec((1, D), lambda i: (0, 0))
    return pl.pallas_call(
        body,
        grid=(N // BN,),
        in_specs=[row, row, row, att, att, wfull, wfull, bfull,
                  pl.BlockSpec((H, D), lambda i: (0, 0)), bfull, bfull],
        out_specs=row,
        out_shape=jax.ShapeDtypeStruct((N, D), jnp.float32),
    )(dst_x, num0, num1, den0, den1, Wo1, B2, c2.reshape(1, D), R,
      gamma.reshape(1, D), beta.reshape(1, D))


def kernel(src_x, dst_x, edge_index, edge_attr, Wsrc, bsrc, Wdst, bdst, We, be,
           Wattn, battn, Wmsg, bmsg, Wmerge, bmerge, Wout, bout, gamma, beta):
    BD = jax.scipy.linalg.block_diag(*([Wmsg] * H))
    d_ar = jnp.arange(D)
    j_ar = jnp.arange(L16)
    headmask = (d_ar[:, None] // HD) == (j_ar[None, :] % 4)

    def attn_fold(off):
        return jnp.where(headmask, Wattn[off + (d_ar % HD), 0][:, None], 0.0)
    A_s, A_d, A_e = attn_fold(0), attn_fold(HD), attn_fold(2 * HD)
    WsBD = Wsrc @ BD
    bsBD = bsrc @ BD
    WeBD = We @ BD
    beBD = be @ BD + jnp.tile(bmsg, H)
    bat = jnp.broadcast_to(battn[0], (1, L16)).astype(jnp.float32)
    R = jnp.where(jnp.arange(H)[:, None] == d_ar[None, :] // HD, 1.0, 0.0)
    Wm_t = jnp.tile(Wmerge, (H, 1)) / H
    B2 = Wm_t @ Wout[D:]
    c2 = bmerge @ Wout[D:] + bout

    src_idx, dst_idx = edge_index[0], edge_index[1]
    didx32n = N_PAD + dst_idx // 32
    qblk = (jnp.arange(L16) // 4).astype(jnp.float32)
    dcmp16 = (dst_idx % 32).astype(jnp.float32)[:, None] - qblk[None, :]
    st = _node_tc(src_x, Wsrc, bsrc, WsBD, bsBD, A_s)
    epack = _edge_tc(edge_attr, dcmp16, We, be, WeBD, beBD, A_e, bat)
    G = E // C
    idxcat = jnp.concatenate(
        [dst_idx.reshape(G, C), didx32n.reshape(G, C)], axis=1).reshape(-1)

    num_p, den_p = _sc_edge(src_idx, idxcat, epack, st)
    den_r = den_p.reshape(NC, DG * 32, H)

    return _tail_tc(dst_x, num_p[0, :N], num_p[1, :N],
                    den_r[0, :N], den_r[1, :N],
                    Wout[:D], B2, c2, R, gamma, beta)


# speedup vs baseline: 4.1687x; 1.0523x over previous
import functools

import jax
import jax.numpy as jnp
from jax import lax
from jax.experimental import pallas as pl
from jax.experimental.pallas import tpu as pltpu
from jax.experimental.pallas import tpu_sc as plsc

N = 10000
E = 320000
D = 128
H = 4
HD = 32
L16 = 16

NC = 2
NS = 16
C = 40
TPE = E // (NC * NS)
CHUNKS = TPE // C
RPT = 632
N_PAD = NS * RPT
RQ, RTAIL = RPT // C, RPT % C
DG = 320
ACC = 10496


def _node_tc(src_x, Wsrc, bsrc, WsBD, bsBD, A_s):
    BN = 2000

    def body(sx, ws, bs, wsbd, bsbd, a_s, st_o):
        x = sx[...]
        sh = jnp.dot(x, ws[...], preferred_element_type=jnp.float32) + bs[...]
        sm = jnp.dot(x, wsbd[...], preferred_element_type=jnp.float32) + bsbd[...]
        g = jnp.exp(jnp.dot(jnp.tanh(sh), a_s[...], preferred_element_type=jnp.float32))
        st_o[...] = jnp.concatenate(
            [sm, g, jnp.zeros((BN, D - L16), jnp.float32)], axis=-1)

    row = pl.BlockSpec((BN, D), lambda i: (i, 0))
    wfull = pl.BlockSpec((D, D), lambda i: (0, 0))
    bfull = pl.BlockSpec((1, D), lambda i: (0, 0))
    afull = pl.BlockSpec((D, L16), lambda i: (0, 0))
    return pl.pallas_call(
        body,
        grid=(N // BN,),
        in_specs=[row, wfull, bfull, wfull, bfull, afull],
        out_specs=pl.BlockSpec((BN, 2 * D), lambda i: (i, 0)),
        out_shape=jax.ShapeDtypeStruct((N, 2 * D), jnp.float32),
    )(src_x, Wsrc, bsrc.reshape(1, D), WsBD, bsBD.reshape(1, D), A_s)


def _edge_tc(edge_attr, dmod16, We, be, WeBD, beBD, A_e, bat):
    BE = 8000
    ED = edge_attr.shape[1]

    def body(ea, dm, we, b_e, webd, bebd, a_e, bt, ep_o):
        a = ea[...]
        ef = jnp.dot(a, we[...], preferred_element_type=jnp.float32) + b_e[...]
        em = jnp.dot(a, webd[...], preferred_element_type=jnp.float32) + bebd[...]
        eea = jnp.exp(jnp.dot(jnp.tanh(ef), a_e[...],
                              preferred_element_type=jnp.float32) + bt[...])
        ep_o[...] = jnp.concatenate([em, eea, dm[...]], axis=-1)

    row = pl.BlockSpec((BE, ED), lambda i: (i, 0))
    return pl.pallas_call(
        body,
        grid=(E // BE,),
        in_specs=[row,
                  pl.BlockSpec((BE, L16), lambda i: (i, 0)),
                  pl.BlockSpec((ED, D), lambda i: (0, 0)),
                  pl.BlockSpec((1, D), lambda i: (0, 0)),
                  pl.BlockSpec((ED, D), lambda i: (0, 0)),
                  pl.BlockSpec((1, D), lambda i: (0, 0)),
                  pl.BlockSpec((D, L16), lambda i: (0, 0)),
                  pl.BlockSpec((1, L16), lambda i: (0, 0))],
        out_specs=pl.BlockSpec((BE, D + 2 * L16), lambda i: (i, 0)),
        out_shape=jax.ShapeDtypeStruct((E, D + 2 * L16), jnp.float32),
    )(edge_attr, dmod16, We, be.reshape(1, D), WeBD, beBD.reshape(1, D), A_e, bat)


def _sc_edge(src_idx, idxcat, epack, st):
    mesh = plsc.VectorSubcoreMesh(core_axis_name="c", subcore_axis_name="s")
    EPW = D + 2 * L16

    @functools.partial(
        pl.kernel,
        out_type=(jax.ShapeDtypeStruct((NC, N_PAD, D), jnp.float32),
                  jax.ShapeDtypeStruct((NC, DG, D), jnp.float32)),
        mesh=mesh,
        scratch_types=[
            pltpu.VMEM((C,), jnp.int32),
            pltpu.VMEM((C,), jnp.int32),
            pltpu.VMEM((2 * C,), jnp.int32),
            pltpu.VMEM((2 * C,), jnp.int32),
            pltpu.VMEM((C, EPW), jnp.float32),
            pltpu.VMEM((C, 2 * D), jnp.float32),
            pltpu.VMEM((C, 2 * D), jnp.float32),
            pltpu.VMEM((2 * C, D), jnp.float32),
            pltpu.VMEM_SHARED((ACC, D), jnp.float32),
            pltpu.SemaphoreType.DMA,
            pltpu.SemaphoreType.DMA,
            pltpu.SemaphoreType.DMA,
            pltpu.SemaphoreType.DMA,
            pltpu.SemaphoreType.DMA,
        ],
    )
    def k(sidx_hbm, idxcat_hbm, ep_hbm, st_hbm,
          num_out, den_out,
          sidx_v0, sidx_v1, idxcat_v0, idxcat_v1, ep_v, st_v, st_v1, wd_v,
          acc_sh, sem_a, sem_b, sem_g0, sem_g1, sem_e):
        c = lax.axis_index("c")
        s = lax.axis_index("s")
        zero16 = jnp.zeros((L16,), jnp.float32)

        def zrow(i, cc):
            for l in range(D // L16):
                wd_v[i, pl.ds(l * L16, L16)] = zero16
            return cc
        lax.fori_loop(0, 2 * C, zrow, 0)
        acc_pt = ACC // NS
        zbase = s * acc_pt
        for q in range(acc_pt // (2 * C)):
            pltpu.sync_copy(wd_v, acc_sh.at[pl.ds(zbase + q * 2 * C, 2 * C)])
        zt = acc_pt - (acc_pt // (2 * C)) * (2 * C)
        if zt:
            pltpu.sync_copy(wd_v.at[pl.ds(0, zt)],
                            acc_sh.at[pl.ds(zbase + acc_pt - zt, zt)])
        plsc.subcore_barrier()

        tbase = (c * NS + s) * CHUNKS
        NPAIR = CHUNKS // 2

        def issue_wave(ci, sv, iv, sem):
            off = (tbase + ci) * C
            pltpu.async_copy(sidx_hbm.at[pl.ds(off, C)], sv, sem)
            pltpu.async_copy(idxcat_hbm.at[pl.ds(2 * off, 2 * C)], iv, sem)

        def drain_wave(sv, iv, sem):
            pltpu.make_async_copy(sidx_hbm.at[pl.ds(0, C)], sv, sem).wait()
            pltpu.make_async_copy(idxcat_hbm.at[pl.ds(0, 2 * C)], iv, sem).wait()

        def issue_ep(ci):
            pltpu.async_copy(ep_hbm.at[pl.ds((tbase + ci) * C, C)], ep_v, sem_e)

        def drain_ep():
            pltpu.make_async_copy(ep_hbm.at[pl.ds(0, C)], ep_v, sem_e).wait()

        def compute(stv, iv):
            def row(i, rc):
                ex16 = stv[i, pl.ds(D, L16)] * ep_v[i, pl.ds(D, L16)]
                dm = ep_v[i, pl.ds(D + L16, L16)]
                for h in range(H):
                    spl = jnp.broadcast_to(ex16[h], (L16,))
                    for l in (2 * h, 2 * h + 1):
                        sl = pl.ds(l * L16, L16)
                        wd_v[i, sl] = (stv[i, sl] + ep_v[i, sl]) * spl
                for l in range(D // L16):
                    wd_v[C + i, pl.ds(l * L16, L16)] = jnp.where(
                        dm == float(4 * l), ex16, 0.0)
                return rc
            lax.fori_loop(0, C, row, 0, unroll=4)
            pltpu.sync_copy(wd_v, acc_sh.at[iv], add=True)

        issue_wave(0, sidx_v0, idxcat_v0, sem_a)
        issue_wave(1, sidx_v1, idxcat_v1, sem_b)
        drain_wave(sidx_v0, idxcat_v0, sem_a)
        pltpu.async_copy(st_hbm.at[sidx_v0], st_v, sem_g0)
        issue_ep(0)

        def pair(p, cc):
            drain_wave(sidx_v1, idxcat_v1, sem_b)
            pltpu.async_copy(st_hbm.at[sidx_v1], st_v1, sem_g1)
            pltpu.make_async_copy(st_hbm.at[pl.ds(0, C)], st_v, sem_g0).wait()
            drain_ep()
            compute(st_v, idxcat_v0)
            issue_ep(2 * p + 1)

            @pl.when(p < NPAIR - 1)
            def _():
                issue_wave(2 * p + 2, sidx_v0, idxcat_v0, sem_a)
            @pl.when(p < NPAIR - 1)
            def _():
                drain_wave(sidx_v0, idxcat_v0, sem_a)
                pltpu.async_copy(st_hbm.at[sidx_v0], st_v, sem_g0)
            pltpu.make_async_copy(st_hbm.at[pl.ds(0, C)], st_v1, sem_g1).wait()
            drain_ep()
            compute(st_v1, idxcat_v1)

            @pl.when(p < NPAIR - 1)
            def _():
                issue_ep(2 * p + 2)
                issue_wave(2 * p + 3, sidx_v1, idxcat_v1, sem_b)
            return cc
        lax.fori_loop(0, NPAIR, pair, 0)
        plsc.subcore_barrier()

        nbase = s * RPT
        for q in range(RPT // (2 * C)):
            pltpu.sync_copy(acc_sh.at[pl.ds(nbase + q * 2 * C, 2 * C)], wd_v)
            pltpu.sync_copy(wd_v, num_out.at[c, pl.ds(nbase + q * 2 * C, 2 * C)])
        rt = RPT - (RPT // (2 * C)) * (2 * C)
        if rt:
            pltpu.sync_copy(acc_sh.at[pl.ds(nbase + RPT - rt, rt)],
                            wd_v.at[pl.ds(0, rt)])
            pltpu.sync_copy(wd_v.at[pl.ds(0, rt)],
                            num_out.at[c, pl.ds(nbase + RPT - rt, rt)])
        @pl.when(s < DG // (2 * C))
        def _():
            dbase = s * 2 * C
            pltpu.sync_copy(acc_sh.at[pl.ds(N_PAD + dbase, 2 * C)], wd_v)
            pltpu.sync_copy(wd_v, den_out.at[c, pl.ds(dbase, 2 * C)])

    return k(src_idx, idxcat, epack, st)


def _tail_tc(dst_x, num0, num1, den0, den1, Wo1, B2, c2, R, gamma, beta):
    BN = 2000

    def body(dx, n0, n1, d0, d1, wo1, b2, c2r, r, g, b, out):
        x = dx[...]
        num = n0[...] + n1[...]
        den = d0[...] + d1[...]
        den_bc = jnp.dot(den, r[...], preferred_element_type=jnp.float32)
        agg = num / jnp.where(den_bc > 0, den_bc, 1.0)
        res = (x + jnp.dot(x, wo1[...], preferred_element_type=jnp.float32)
               + jnp.dot(agg, b2[...], preferred_element_type=jnp.float32) + c2r[...])
        mu = jnp.mean(res, axis=-1, keepdims=True)
        cen = res - mu
        var = jnp.mean(cen * cen, axis=-1, keepdims=True)
        out[...] = cen * jax.lax.rsqrt(var + 1e-5) * g[...] + b[...]

    row = pl.BlockSpec((BN, D), lambda i: (i, 0))
    att = pl.BlockSpec((BN, H), lambda i: (i, 0))
    wfull = pl.BlockSpec((D, D), lambda i: (0, 0))
    bfull = pl.BlockSpec((1, D), lambda i: (0, 0))
    return pl.pallas_call(
        body,
        grid=(N // BN,),
        in_specs=[row, row, row, att, att, wfull, wfull, bfull,
                  pl.BlockSpec((H, D), lambda i: (0, 0)), bfull, bfull],
        out_specs=row,
        out_shape=jax.ShapeDtypeStruct((N, D), jnp.float32),
    )(dst_x, num0, num1, den0, den1, Wo1, B2, c2.reshape(1, D), R,
      gamma.reshape(1, D), beta.reshape(1, D))


def kernel(src_x, dst_x, edge_index, edge_attr, Wsrc, bsrc, Wdst, bdst, We, be,
           Wattn, battn, Wmsg, bmsg, Wmerge, bmerge, Wout, bout, gamma, beta):
    BD = jax.scipy.linalg.block_diag(*([Wmsg] * H))
    d_ar = jnp.arange(D)
    j_ar = jnp.arange(L16)
    headmask = (d_ar[:, None] // HD) == (j_ar[None, :] % 4)

    def attn_fold(off):
        return jnp.where(headmask, Wattn[off + (d_ar % HD), 0][:, None], 0.0)
    A_s, A_d, A_e = attn_fold(0), attn_fold(HD), attn_fold(2 * HD)
    WsBD = Wsrc @ BD
    bsBD = bsrc @ BD
    WeBD = We @ BD
    beBD = be @ BD + jnp.tile(bmsg, H)
    bat = jnp.broadcast_to(battn[0], (1, L16)).astype(jnp.float32)
    R = jnp.where(jnp.arange(H)[:, None] == d_ar[None, :] // HD, 1.0, 0.0)
    Wm_t = jnp.tile(Wmerge, (H, 1)) / H
    B2 = Wm_t @ Wout[D:]
    c2 = bmerge @ Wout[D:] + bout

    src_idx, dst_idx = edge_index[0], edge_index[1]
    didx32n = N_PAD + dst_idx // 32
    qblk = (jnp.arange(L16) // 4).astype(jnp.float32)
    dcmp16 = (dst_idx % 32).astype(jnp.float32)[:, None] - qblk[None, :]
    st = _node_tc(src_x, Wsrc, bsrc, WsBD, bsBD, A_s)
    epack = _edge_tc(edge_attr, dcmp16, We, be, WeBD, beBD, A_e, bat)
    G = E // C
    idxcat = jnp.concatenate(
        [dst_idx.reshape(G, C), didx32n.reshape(G, C)], axis=1).reshape(-1)

    num_p, den_p = _sc_edge(src_idx, idxcat, epack, st)
    den_r = den_p.reshape(NC, DG * 32, H)

    return _tail_tc(dst_x, num_p[0, :N], num_p[1, :N],
                    den_r[0, :N], den_r[1, :N],
                    Wout[:D], B2, c2, R, gamma, beta)
